# Initial kernel scaffold; baseline (speedup 1.0000x reference)
import functools

import jax
import jax.numpy as jnp
from jax import lax
from jax.experimental import pallas as pl
from jax.experimental.pallas import tpu as pltpu
from jax.experimental.pallas import tpu_sc as plsc

_N = 10000
_F = 128
_E = 320000
_G = 64
_NC = 2
_NS = 16
_NW = _NC * _NS
_IR = 128
_ROWS = _E // _IR
_RPS = _N // _NS


def _vmesh():
    return plsc.VectorSubcoreMesh(core_axis_name="c", subcore_axis_name="s")


def _sc_degree(dst2d, ones_blk, zero_blk):
    @functools.partial(
        pl.kernel,
        out_type=jax.ShapeDtypeStruct((_NC, _N, 16), jnp.float32),
        mesh=_vmesh(),
        scratch_types=[
            pltpu.VMEM((_IR,), jnp.int32),
            pltpu.VMEM((_IR, 16), jnp.float32),
            pltpu.VMEM_SHARED((_N, 16), jnp.float32),
        ],
    )
    def hist_kernel(dst_hbm, ones_hbm, z_hbm, out_hbm, dst_v, ones_v, hist_sh):
        cid = lax.axis_index("c")
        sid = lax.axis_index("s")
        w = cid * _NS + sid
        row0 = sid * _RPS
        pltpu.sync_copy(ones_hbm, ones_v)
        pltpu.sync_copy(z_hbm, hist_sh.at[pl.ds(row0, _RPS)])
        plsc.subcore_barrier()
        start = w * _ROWS // _NW
        stop = (w + 1) * _ROWS // _NW

        @pl.loop(start, stop)
        def _(j):
            pltpu.sync_copy(dst_hbm.at[j], dst_v)
            pltpu.sync_copy(ones_v, hist_sh.at[dst_v], add=True)

        plsc.subcore_barrier()
        pltpu.sync_copy(hist_sh.at[pl.ds(row0, _RPS)],
                        out_hbm.at[cid, pl.ds(row0, _RPS)])

    return hist_kernel(dst2d, ones_blk, zero_blk)


def _sc_scatter_add(p, src2d, dst2d, zero_blk):
    @functools.partial(
        pl.kernel,
        out_type=jax.ShapeDtypeStruct((_NC, _N, _F), jnp.float32),
        mesh=_vmesh(),
        scratch_types=[
            pltpu.VMEM((_IR,), jnp.int32),
            pltpu.VMEM((_IR,), jnp.int32),
            pltpu.VMEM((_IR, _F), jnp.float32),
            pltpu.VMEM_SHARED((_N, _F), jnp.float32),
            pltpu.SemaphoreType.DMA,
        ],
    )
    def scatter_kernel(p_hbm, src_hbm, dst_hbm, z_hbm, out_hbm,
                       src_v, dst_v, rows_v, acc_sh, sem):
        cid = lax.axis_index("c")
        sid = lax.axis_index("s")
        w = cid * _NS + sid
        row0 = sid * _RPS
        pltpu.sync_copy(z_hbm, acc_sh.at[pl.ds(row0, _RPS)])
        plsc.subcore_barrier()
        start = w * _ROWS // _NW
        stop = (w + 1) * _ROWS // _NW

        @pl.loop(start, stop)
        def _(j):
            pltpu.sync_copy(src_hbm.at[j], src_v)
            pltpu.sync_copy(dst_hbm.at[j], dst_v)
            pltpu.async_copy(p_hbm.at[src_v], rows_v, sem).wait()
            pltpu.sync_copy(rows_v, acc_sh.at[dst_v], add=True)

        plsc.subcore_barrier()
        pltpu.sync_copy(acc_sh.at[pl.ds(row0, _RPS)],
                        out_hbm.at[cid, pl.ds(row0, _RPS)])

    return scatter_kernel(p, src2d, dst2d, zero_blk)


def _mm(a, b):
    return lax.dot_general(a, b, (((1,), (0,)), ((), ())),
                           precision=lax.Precision.HIGHEST,
                           preferred_element_type=jnp.float32)


def _dis_from_hist(hist_ref):
    deg = hist_ref[0, :, 0:1] + hist_ref[1, :, 0:1] + 1.0
    return lax.rsqrt(deg)


def _tc_first(x, W1, hist):
    def body(x_ref, w_ref, hist_ref, p_ref):
        dis = _dis_from_hist(hist_ref)
        p_ref[...] = _mm(x_ref[...], w_ref[...]) * dis

    return pl.pallas_call(
        body, out_shape=jax.ShapeDtypeStruct((_N, _F), jnp.float32),
    )(x, W1, hist)


def _tc_mid(acc, p1, hist, b1, W2):
    def body(acc_ref, p_ref, hist_ref, b_ref, w_ref, o_ref):
        dis = _dis_from_hist(hist_ref)
        h = dis * (acc_ref[0] + acc_ref[1] + p_ref[...]) + b_ref[...]
        h = jnp.maximum(h, 0.0)
        o_ref[...] = _mm(h, w_ref[...]) * dis

    return pl.pallas_call(
        body, out_shape=jax.ShapeDtypeStruct((_N, _F), jnp.float32),
    )(acc, p1, hist, b1, W2)


def _tc_head(acc, p2, hist, b2, batch2d, L1W, L1b, L2Wp, L2bp):
    def body(acc_ref, p_ref, hist_ref, b_ref, batch_ref,
             l1w_ref, l1b_ref, l2w_ref, l2b_ref, o_ref):
        dis = _dis_from_hist(hist_ref)
        h = dis * (acc_ref[0] + acc_ref[1] + p_ref[...]) + b_ref[...]
        h = jnp.maximum(h, 0.0)
        gid = lax.broadcasted_iota(jnp.int32, (_N, _G), 1)
        mask = jnp.where(batch_ref[...] == gid, 1.0, 0.0)
        sums = lax.dot_general(mask, h, (((0,), (0,)), ((), ())),
                               precision=lax.Precision.HIGHEST,
                               preferred_element_type=jnp.float32)
        cnts = lax.dot_general(mask, jnp.ones((_N, 1), jnp.float32),
                               (((0,), (0,)), ((), ())),
                               precision=lax.Precision.HIGHEST,
                               preferred_element_type=jnp.float32)
        mean = sums / jnp.maximum(cnts, 1.0)
        g = jnp.maximum(_mm(mean, l1w_ref[...]) + l1b_ref[...], 0.0)
        o_ref[...] = _mm(g, l2w_ref[...]) + l2b_ref[...]

    return pl.pallas_call(
        body, out_shape=jax.ShapeDtypeStruct((_G, _F), jnp.float32),
    )(acc, p2, hist, b2, batch2d, L1W, L1b, L2Wp, L2bp)


def kernel(x, edge_index, batch, W1, b1, W2, b2, L1W, L1b, L2W, L2b):
    src2d = edge_index[0].reshape(_ROWS, _IR)
    dst2d = edge_index[1].reshape(_ROWS, _IR)
    batch2d = batch.reshape(_N, 1)
    ones_blk = jnp.ones((_IR, 16), jnp.float32)
    zero16 = jnp.zeros((_RPS, 16), jnp.float32)
    zrows = jnp.zeros((_RPS, _F), jnp.float32)
    b1r = b1.reshape(1, _F)
    b2r = b2.reshape(1, _F)
    L1br = L1b.reshape(1, 64)
    L2Wp = jnp.pad(L2W, ((0, 0), (0, _F - 40)))
    L2bp = jnp.pad(L2b, (0, _F - 40)).reshape(1, _F)

    hist = _sc_degree(dst2d, ones_blk, zero16)
    p1 = _tc_first(x, W1, hist)
    acc1 = _sc_scatter_add(p1, src2d, dst2d, zrows)
    p2 = _tc_mid(acc1, p1, hist, b1r, W2)
    acc2 = _sc_scatter_add(p2, src2d, dst2d, zrows)
    outp = _tc_head(acc2, p2, hist, b2r, batch2d, L1W, L1br, L2Wp, L2bp)
    return outp[:, :40]


# speedup vs baseline: 15.8870x; 15.8870x over previous
import functools

import jax
import jax.numpy as jnp
from jax import lax
from jax.experimental import pallas as pl
from jax.experimental.pallas import tpu as pltpu
from jax.experimental.pallas import tpu_sc as plsc

_N = 10000
_F = 128
_E = 320000
_G = 64
_NC = 2
_NS = 16
_NW = _NC * _NS
_IR = 128
_ROWS = _E // _IR
_RB = 1000
_NRB = _N // _RB


def _vmesh():
    return plsc.VectorSubcoreMesh(core_axis_name="c", subcore_axis_name="s")


def _sc_degree(dst2d, ones_blk, zero_blk):
    @functools.partial(
        pl.kernel,
        out_type=jax.ShapeDtypeStruct((_NC, _N, 16), jnp.float32),
        mesh=_vmesh(),
        scratch_types=[
            pltpu.VMEM((_IR,), jnp.int32),
            pltpu.VMEM((_IR, 16), jnp.float32),
            pltpu.VMEM_SHARED((_N, 16), jnp.float32),
        ],
    )
    def hist_kernel(dst_hbm, ones_hbm, z_hbm, out_hbm, dst_v, ones_v, hist_sh):
        cid = lax.axis_index("c")
        sid = lax.axis_index("s")
        w = cid * _NS + sid
        row0 = sid * _RB
        pltpu.sync_copy(ones_hbm, ones_v)

        @pl.when(sid < _NRB)
        def _():
            pltpu.sync_copy(z_hbm, hist_sh.at[pl.ds(row0, _RB)])

        plsc.subcore_barrier()
        start = w * _ROWS // _NW
        stop = (w + 1) * _ROWS // _NW

        @pl.loop(start, stop)
        def _(j):
            pltpu.sync_copy(dst_hbm.at[j], dst_v)
            pltpu.sync_copy(ones_v, hist_sh.at[dst_v], add=True)

        plsc.subcore_barrier()

        @pl.when(sid < _NRB)
        def _():
            pltpu.sync_copy(hist_sh.at[pl.ds(row0, _RB)],
                            out_hbm.at[cid, pl.ds(row0, _RB)])

    return hist_kernel(dst2d, ones_blk, zero_blk)


def _sc_scatter_add(p, src2d, dst2d, zero_blk):
    @functools.partial(
        pl.kernel,
        out_type=jax.ShapeDtypeStruct((_NC, _N, _F), jnp.float32),
        mesh=_vmesh(),
        scratch_types=[
            pltpu.VMEM((_IR,), jnp.int32),
            pltpu.VMEM((_IR,), jnp.int32),
            pltpu.VMEM((_IR, _F), jnp.float32),
            pltpu.VMEM_SHARED((_N, _F), jnp.float32),
            pltpu.SemaphoreType.DMA,
        ],
    )
    def scatter_kernel(p_hbm, src_hbm, dst_hbm, z_hbm, out_hbm,
                       src_v, dst_v, rows_v, acc_sh, sem):
        cid = lax.axis_index("c")
        sid = lax.axis_index("s")
        w = cid * _NS + sid
        row0 = sid * _RB

        @pl.when(sid < _NRB)
        def _():
            pltpu.sync_copy(z_hbm, acc_sh.at[pl.ds(row0, _RB)])

        plsc.subcore_barrier()
        start = w * _ROWS // _NW
        stop = (w + 1) * _ROWS // _NW

        @pl.loop(start, stop)
        def _(j):
            pltpu.sync_copy(src_hbm.at[j], src_v)
            pltpu.sync_copy(dst_hbm.at[j], dst_v)
            pltpu.async_copy(p_hbm.at[src_v], rows_v, sem).wait()
            pltpu.sync_copy(rows_v, acc_sh.at[dst_v], add=True)

        plsc.subcore_barrier()

        @pl.when(sid < _NRB)
        def _():
            pltpu.sync_copy(acc_sh.at[pl.ds(row0, _RB)],
                            out_hbm.at[cid, pl.ds(row0, _RB)])

    return scatter_kernel(p, src2d, dst2d, zero_blk)


def _mm(a, b):
    return lax.dot_general(a, b, (((1,), (0,)), ((), ())),
                           precision=lax.Precision.HIGHEST,
                           preferred_element_type=jnp.float32)


def _dis_from_hist(hist_ref):
    deg = hist_ref[0, :, 0:1] + hist_ref[1, :, 0:1] + 1.0
    return lax.rsqrt(deg)


def _tc_first(x, W1, hist):
    def body(x_ref, w_ref, hist_ref, p_ref):
        dis = _dis_from_hist(hist_ref)
        p_ref[...] = _mm(x_ref[...], w_ref[...]) * dis

    return pl.pallas_call(
        body, out_shape=jax.ShapeDtypeStruct((_N, _F), jnp.float32),
    )(x, W1, hist)


def _tc_mid(acc, p1, hist, b1, W2):
    def body(acc_ref, p_ref, hist_ref, b_ref, w_ref, o_ref):
        dis = _dis_from_hist(hist_ref)
        h = dis * (acc_ref[0] + acc_ref[1] + p_ref[...]) + b_ref[...]
        h = jnp.maximum(h, 0.0)
        o_ref[...] = _mm(h, w_ref[...]) * dis

    return pl.pallas_call(
        body, out_shape=jax.ShapeDtypeStruct((_N, _F), jnp.float32),
    )(acc, p1, hist, b1, W2)


def _tc_head(acc, p2, hist, b2, batch2d, L1W, L1b, L2Wp, L2bp):
    def body(acc_ref, p_ref, hist_ref, b_ref, batch_ref,
             l1w_ref, l1b_ref, l2w_ref, l2b_ref, o_ref):
        dis = _dis_from_hist(hist_ref)
        h = dis * (acc_ref[0] + acc_ref[1] + p_ref[...]) + b_ref[...]
        h = jnp.maximum(h, 0.0)
        gid = lax.broadcasted_iota(jnp.int32, (_N, _G), 1)
        mask = jnp.where(batch_ref[...] == gid, 1.0, 0.0)
        sums = lax.dot_general(mask, h, (((0,), (0,)), ((), ())),
                               precision=lax.Precision.HIGHEST,
                               preferred_element_type=jnp.float32)
        cnts = lax.dot_general(mask, jnp.ones((_N, 1), jnp.float32),
                               (((0,), (0,)), ((), ())),
                               precision=lax.Precision.HIGHEST,
                               preferred_element_type=jnp.float32)
        mean = sums / jnp.maximum(cnts, 1.0)
        g = jnp.maximum(_mm(mean, l1w_ref[...]) + l1b_ref[...], 0.0)
        o_ref[...] = _mm(g, l2w_ref[...]) + l2b_ref[...]

    return pl.pallas_call(
        body, out_shape=jax.ShapeDtypeStruct((_G, _F), jnp.float32),
    )(acc, p2, hist, b2, batch2d, L1W, L1b, L2Wp, L2bp)


def kernel(x, edge_index, batch, W1, b1, W2, b2, L1W, L1b, L2W, L2b):
    src2d = edge_index[0].reshape(_ROWS, _IR)
    dst2d = edge_index[1].reshape(_ROWS, _IR)
    batch2d = batch.reshape(_N, 1)
    ones_blk = jnp.ones((_IR, 16), jnp.float32)
    zero16 = jnp.zeros((_RB, 16), jnp.float32)
    zrows = jnp.zeros((_RB, _F), jnp.float32)
    b1r = b1.reshape(1, _F)
    b2r = b2.reshape(1, _F)
    L1br = L1b.reshape(1, 64)
    L2Wp = jnp.pad(L2W, ((0, 0), (0, _F - 40)))
    L2bp = jnp.pad(L2b, (0, _F - 40)).reshape(1, _F)

    hist = _sc_degree(dst2d, ones_blk, zero16)
    p1 = _tc_first(x, W1, hist)
    acc1 = _sc_scatter_add(p1, src2d, dst2d, zrows)
    p2 = _tc_mid(acc1, p1, hist, b1r, W2)
    acc2 = _sc_scatter_add(p2, src2d, dst2d, zrows)
    outp = _tc_head(acc2, p2, hist, b2r, batch2d, L1W, L1br, L2Wp, L2bp)
    return outp[:, :40]
